# baseline (speedup 1.0000x reference)
import functools

import jax
import jax.numpy as jnp
from jax.experimental import pallas as pl
from jax.experimental.pallas import tpu as pltpu

N_TIME = 16
N_NODE = 2048
D_FEAT = 128
BM = 256


_SELU_ALPHA = 1.6732632423543772
_SELU_SCALE = 1.0507009873554805


def _selu(x):
    return _SELU_SCALE * jnp.where(
        x > 0, x, _SELU_ALPHA * (jnp.exp(jnp.minimum(x, 0.0)) - 1.0)
    )


def _gcn_body(a_ref, x_ref, w_ref, o_ref):
    h = jnp.dot(a_ref[0], x_ref[0], preferred_element_type=jnp.float32)
    h = jnp.dot(h, w_ref[...], preferred_element_type=jnp.float32)
    o_ref[...] = _selu(h)


@jax.jit
def kernel(Feature_tensor, adj, W):
    n_time, n_node, d = Feature_tensor.shape
    out_flat = pl.pallas_call(
        _gcn_body,
        grid=(n_time, n_node // BM),
        in_specs=[
            pl.BlockSpec((1, BM, n_node), lambda t, i: (t, i, 0)),
            pl.BlockSpec((1, n_node, d), lambda t, i: (t, 0, 0)),
            pl.BlockSpec((d, d), lambda t, i: (0, 0)),
        ],
        out_specs=pl.BlockSpec((BM, d), lambda t, i: (i, t)),
        out_shape=jax.ShapeDtypeStruct((n_node, n_time * d), jnp.float32),
        compiler_params=pltpu.CompilerParams(
            dimension_semantics=("arbitrary", "arbitrary"),
        ),
    )(adj, Feature_tensor, W)
    return out_flat.reshape(n_node, n_time, d)


# speedup vs baseline: 1.3303x; 1.3303x over previous
import functools

import jax
import jax.numpy as jnp
from jax.experimental import pallas as pl
from jax.experimental.pallas import tpu as pltpu

N_TIME = 16
N_NODE = 2048
D_FEAT = 128
BM = 512


_SELU_ALPHA = 1.6732632423543772
_SELU_SCALE = 1.0507009873554805


def _selu(x):
    return _SELU_SCALE * jnp.where(
        x > 0, x, _SELU_ALPHA * (jnp.exp(jnp.minimum(x, 0.0)) - 1.0)
    )


def _gcn_body(a_ref, x_ref, w_ref, o_ref):
    h = jnp.dot(a_ref[0], x_ref[0], preferred_element_type=jnp.float32)
    h = jnp.dot(h, w_ref[...], preferred_element_type=jnp.float32)
    o_ref[...] = _selu(h)


@jax.jit
def kernel(Feature_tensor, adj, W):
    n_time, n_node, d = Feature_tensor.shape
    out_flat = pl.pallas_call(
        _gcn_body,
        grid=(n_time, n_node // BM),
        in_specs=[
            pl.BlockSpec((1, BM, n_node), lambda t, i: (t, i, 0)),
            pl.BlockSpec((1, n_node, d), lambda t, i: (t, 0, 0)),
            pl.BlockSpec((d, d), lambda t, i: (0, 0)),
        ],
        out_specs=pl.BlockSpec((BM, d), lambda t, i: (i, t)),
        out_shape=jax.ShapeDtypeStruct((n_node, n_time * d), jnp.float32),
        compiler_params=pltpu.CompilerParams(
            dimension_semantics=("arbitrary", "arbitrary"),
        ),
    )(adj, Feature_tensor, W)
    return out_flat.reshape(n_node, n_time, d)


# speedup vs baseline: 1.5439x; 1.1606x over previous
import functools

import jax
import jax.numpy as jnp
from jax.experimental import pallas as pl
from jax.experimental.pallas import tpu as pltpu

N_TIME = 16
N_NODE = 2048
D_FEAT = 128
BM = 1024


_SELU_ALPHA = 1.6732632423543772
_SELU_SCALE = 1.0507009873554805


def _selu(x):
    return _SELU_SCALE * jnp.where(
        x > 0, x, _SELU_ALPHA * (jnp.exp(jnp.minimum(x, 0.0)) - 1.0)
    )


def _gcn_body(a_ref, x_ref, w_ref, o_ref):
    h = jnp.dot(a_ref[0], x_ref[0], preferred_element_type=jnp.float32)
    h = jnp.dot(h, w_ref[...], preferred_element_type=jnp.float32)
    o_ref[...] = _selu(h)


@jax.jit
def kernel(Feature_tensor, adj, W):
    n_time, n_node, d = Feature_tensor.shape
    out_flat = pl.pallas_call(
        _gcn_body,
        grid=(n_time, n_node // BM),
        in_specs=[
            pl.BlockSpec((1, BM, n_node), lambda t, i: (t, i, 0)),
            pl.BlockSpec((1, n_node, d), lambda t, i: (t, 0, 0)),
            pl.BlockSpec((d, d), lambda t, i: (0, 0)),
        ],
        out_specs=pl.BlockSpec((BM, d), lambda t, i: (i, t)),
        out_shape=jax.ShapeDtypeStruct((n_node, n_time * d), jnp.float32),
        compiler_params=pltpu.CompilerParams(
            dimension_semantics=("arbitrary", "arbitrary"),
        ),
    )(adj, Feature_tensor, W)
    return out_flat.reshape(n_node, n_time, d)


# speedup vs baseline: 1.5560x; 1.0078x over previous
import functools

import jax
import jax.numpy as jnp
from jax.experimental import pallas as pl
from jax.experimental.pallas import tpu as pltpu

N_TIME = 16
N_NODE = 2048
D_FEAT = 128
BM = 2048


_SELU_ALPHA = 1.6732632423543772
_SELU_SCALE = 1.0507009873554805


def _selu(x):
    return _SELU_SCALE * jnp.where(
        x > 0, x, _SELU_ALPHA * (jnp.exp(jnp.minimum(x, 0.0)) - 1.0)
    )


def _gcn_body(a_ref, x_ref, w_ref, o_ref):
    h = jnp.dot(a_ref[0], x_ref[0], preferred_element_type=jnp.float32)
    h = jnp.dot(h, w_ref[...], preferred_element_type=jnp.float32)
    o_ref[...] = _selu(h)


@jax.jit
def kernel(Feature_tensor, adj, W):
    n_time, n_node, d = Feature_tensor.shape
    out_flat = pl.pallas_call(
        _gcn_body,
        grid=(n_time, n_node // BM),
        in_specs=[
            pl.BlockSpec((1, BM, n_node), lambda t, i: (t, i, 0)),
            pl.BlockSpec((1, n_node, d), lambda t, i: (t, 0, 0)),
            pl.BlockSpec((d, d), lambda t, i: (0, 0)),
        ],
        out_specs=pl.BlockSpec((BM, d), lambda t, i: (i, t)),
        out_shape=jax.ShapeDtypeStruct((n_node, n_time * d), jnp.float32),
        compiler_params=pltpu.CompilerParams(
            dimension_semantics=("arbitrary", "arbitrary"),
        ),
    )(adj, Feature_tensor, W)
    return out_flat.reshape(n_node, n_time, d)
